# baseline (speedup 1.0000x reference)
import functools

import jax
import jax.numpy as jnp
from jax import lax
from jax.experimental import pallas as pl
from jax.experimental.pallas import tpu as pltpu
from jax.experimental.pallas import tpu_sc as plsc

_ANCH = ((116.0, 90.0, 156.0, 198.0, 373.0, 326.0),
         (30.0, 61.0, 62.0, 45.0, 59.0, 119.0),
         (10.0, 13.0, 16.0, 30.0, 33.0, 23.0))
_RESO = 416.0
_NWS = (13, 26, 52)
_BASES = (0, 507, 2535)
_NANS = (507, 2028, 8112)
_A = 10647
_NT = 50
_NTP = 64
_NB = 16
_TH = 0.6
_GD = 16


def _iou(x1, y1, w1, h1, x2, y2, w2, h2):
    ax1 = x1 - w1 / 2.0
    ax2 = x1 + w1 / 2.0
    ay1 = y1 - h1 / 2.0
    ay2 = y1 + h1 / 2.0
    bx1 = x2 - w2 / 2.0
    bx2 = x2 + w2 / 2.0
    by1 = y2 - h2 / 2.0
    by2 = y2 + h2 / 2.0
    iw = jnp.maximum(jnp.minimum(ax2, bx2) - jnp.maximum(ax1, bx1), 0.0)
    ih = jnp.maximum(jnp.minimum(ay2, by2) - jnp.maximum(ay1, by1), 0.0)
    inter = iw * ih
    union = (ax2 - ax1) * (ay2 - ay1) + (bx2 - bx1) * (by2 - by1) - inter
    return inter / jnp.maximum(union, 1e-16)


def _best_slot(t1, t2, t3, t4, lvl):
    nw = float(_NWS[lvl])
    nwi = _NWS[lvl]
    nan = _NANS[lvl]
    aw = _ANCH[lvl]
    gw = t3 * nw
    gh = t4 * nw
    i0 = _iou(0.0, 0.0, aw[0], aw[1], 0.0, 0.0, gw, gh)
    i1 = _iou(0.0, 0.0, aw[2], aw[3], 0.0, 0.0, gw, gh)
    i2 = _iou(0.0, 0.0, aw[4], aw[5], 0.0, 0.0, gw, gh)
    bn = jnp.where((i0 >= i1) & (i0 >= i2), 0, jnp.where(i1 >= i2, 1, 2))
    bmax = jnp.maximum(jnp.maximum(i0, i1), i2)
    bn = jnp.where(bmax > 0.0, bn, -1)
    gi = (t1 * nw).astype(jnp.int32)
    gj = (t2 * nw).astype(jnp.int32)
    idx = 3 * (nwi * gi + gj) + bn
    return jnp.where(idx < 0, idx + nan, idx)


def _sc_gather_kernel(tgt_hbm, table_hbm, g_hbm, tv, idxv, rows, sem):
    wid = lax.axis_index("s") * 2 + lax.axis_index("c")

    @pl.when(wid < _NB)
    def _():
        b = wid
        pltpu.sync_copy(tgt_hbm.at[b], tv)
        for lvl in range(3):
            for i in range(_NTP // 16):
                sl = pl.ds(i * 16, 16)
                t1 = tv[1, sl]
                t2 = tv[2, sl]
                t3 = tv[3, sl]
                t4 = tv[4, sl]
                idx = _best_slot(t1, t2, t3, t4, lvl)
                g = idx + (b * _A + _BASES[lvl])
                idxv[lvl, sl] = lax.shift_right_logical(g, 3)
        copies = [
            pltpu.async_copy(table_hbm.at[idxv.at[lvl]], rows.at[lvl], sem)
            for lvl in range(3)
        ]
        for c in copies:
            c.wait()
        pltpu.sync_copy(rows, g_hbm.at[b])


def _sc_gather(trt64, out16):
    mesh = plsc.VectorSubcoreMesh(core_axis_name="c", subcore_axis_name="s")
    run = functools.partial(
        pl.kernel,
        mesh=mesh,
        out_type=jax.ShapeDtypeStruct((_NB, 3, _NTP, 128), jnp.float32),
        scratch_types=[
            pltpu.VMEM((5, _NTP), jnp.float32),
            pltpu.VMEM((3, _NTP), jnp.int32),
            pltpu.VMEM((3, _NTP, 128), jnp.float32),
            pltpu.SemaphoreType.DMA,
        ],
    )(_sc_gather_kernel)
    return run(trt64, out16)


def _body(out_ref, trc_ref, trt_ref, g_ref, loss_ref):
    blk4 = out_ref[0, :, 0:4] / _RESO
    px = blk4[:, 0:1]
    py = blk4[:, 1:2]
    pw = blk4[:, 2:3]
    ph = blk4[:, 3:4]
    base = 0.5 * (jnp.sum((px - 0.5) ** 2) + jnp.sum((py - 0.5) ** 2)
                  + jnp.sum(pw * pw) + jnp.sum(ph * ph))
    cls = out_ref[0, :, 5:85]
    base = base - jnp.sum(jnp.maximum(jnp.log(1.0 - cls), -100.0))
    conf = out_ref[0, :, 5:6]

    t0r = trt_ref[0, 0:1, :]
    t1r = trt_ref[0, 1:2, :]
    t2r = trt_ref[0, 2:3, :]
    t3r = trt_ref[0, 3:4, :]
    t4r = trt_ref[0, 4:5, :]
    trc = trc_ref[0]
    t0c = trc[:, 0:1]
    t1c = trc[:, 1:2]
    t2c = trc[:, 2:3]
    t3c = trc[:, 3:4]
    t4c = trc[:, 4:5]

    ii = lax.broadcasted_iota(jnp.int32, (_NT, _NT), 0)
    jj = lax.broadcasted_iota(jnp.int32, (_NT, _NT), 1)
    zr = (t1r == 0.0).astype(jnp.float32)
    zc = (t1c == 0.0).astype(jnp.float32)
    prefT = jnp.sum(jnp.where(ii <= jj, zc, 0.0), axis=0, keepdims=True)
    aliveT = prefT == 0.0
    prefC = jnp.sum(jnp.where(jj <= ii, zr, 0.0), axis=1, keepdims=True)
    aliveC = prefC == 0.0

    aidx = lax.broadcasted_iota(jnp.int32, (_A, 1), 0)
    nwv = jnp.where(aidx < _BASES[1], float(_NWS[0]),
                    jnp.where(aidx < _BASES[2], float(_NWS[1]), float(_NWS[2])))
    iou_all = _iou(px, py, pw, ph, t1r * nwv, t2r * nwv, t3r * nwv, t4r * nwv)
    iou_all = jnp.where(aliveT, iou_all, 0.0)
    cur = jnp.max(iou_all, axis=1, keepdims=True)
    base = base + jnp.sum(jnp.where(cur > _TH, 0.0, 12.5 * conf * conf))

    iota128 = lax.broadcasted_iota(jnp.int32, (_NT, 128), 1)
    b_id = pl.program_id(0)
    corr = jnp.float32(0.0)
    for lvl in range(3):
        nw = float(_NWS[lvl])
        idxr = _best_slot(t1r, t2r, t3r, t4r, lvl)
        idxc = _best_slot(t1c, t2c, t3c, t4c, lvl)

        eq = idxc == idxr
        later = jnp.sum(jnp.where((jj > ii) & eq & aliveT, 1.0, 0.0),
                        axis=1, keepdims=True)
        winc = aliveC & (later == 0.0)
        keyc = idxc * 128 + t0c.astype(jnp.int32)
        keyr = idxr * 128 + t0r.astype(jnp.int32)
        eqk = keyc == keyr
        earlier = jnp.sum(jnp.where((jj < ii) & eqk & aliveT, 1.0, 0.0),
                          axis=1, keepdims=True)
        firstc = aliveC & (earlier == 0.0)

        g = g_ref[0, lvl, 0:_NT, :]
        colb = ((b_id * _A + _BASES[lvl] + idxc) & 7) * 16

        def _pick(ch):
            return jnp.sum(jnp.where(iota128 == colb + ch, g, 0.0),
                           axis=1, keepdims=True)

        pxs = _pick(0) / _RESO
        pys = _pick(1) / _RESO
        pws = _pick(2) / _RESO
        phs = _pick(3) / _RESO
        confs = _pick(5)
        ccol = t0c.astype(jnp.int32)
        vcls = _pick(5 + ccol)

        dcoord = 0.5 * ((pxs - t1c) ** 2 - (pxs - 0.5) ** 2
                        + (pys - t2c) ** 2 - (pys - 0.5) ** 2
                        + (pws - t3c) ** 2 - pws * pws
                        + (phs - t4c) ** 2 - phs * phs)
        tconf = _iou(t1c * nw, t2c * nw, t3c * nw, t4c * nw, pxs, pys, pws, phs)
        slot_all = _iou(pxs, pys, pws, phs, t1r * nw, t2r * nw, t3r * nw, t4r * nw)
        slot_all = jnp.where(aliveT, slot_all, 0.0)
        curs = jnp.max(slot_all, axis=1, keepdims=True)
        dconf = 0.5 * (confs - tconf) ** 2 - jnp.where(
            curs > _TH, 0.0, 12.5 * confs * confs)
        dcls = (-jnp.maximum(jnp.log(vcls), -100.0)
                + jnp.maximum(jnp.log(1.0 - vcls), -100.0))
        corr = corr + jnp.sum(jnp.where(winc, dcoord + dconf, 0.0))
        corr = corr + jnp.sum(jnp.where(firstc, dcls, 0.0))

    loss_ref[0, 0, 0] = base + corr


def kernel(output, target):
    trc = target.reshape(_NB, _NT, 5)
    trt = jnp.transpose(trc, (0, 2, 1))
    trt64 = jnp.pad(trt, ((0, 0), (0, 0), (0, _NTP - _NT)))
    out16 = output[:, :, 0:_GD].reshape(_NB * _A * _GD // 128, 128)
    g = _sc_gather(trt64, out16)
    partial = pl.pallas_call(
        _body,
        grid=(_NB,),
        in_specs=[
            pl.BlockSpec((1, _A, 85), lambda b: (b, 0, 0)),
            pl.BlockSpec((1, _NT, 5), lambda b: (b, 0, 0)),
            pl.BlockSpec((1, 5, _NT), lambda b: (b, 0, 0)),
            pl.BlockSpec((1, 3, _NTP, 128), lambda b: (b, 0, 0, 0)),
        ],
        out_specs=pl.BlockSpec((1, 1, 1), lambda b: (b, 0, 0),
                               memory_space=pltpu.SMEM),
        out_shape=jax.ShapeDtypeStruct((_NB, 1, 1), jnp.float32),
    )(output, trc, trt, g)
    return jnp.sum(partial)


# speedup vs baseline: 1.9897x; 1.9897x over previous
import functools

import jax
import jax.numpy as jnp
from jax import lax
from jax.experimental import pallas as pl
from jax.experimental.pallas import tpu as pltpu
from jax.experimental.pallas import tpu_sc as plsc

_ANCH = ((116.0, 90.0, 156.0, 198.0, 373.0, 326.0),
         (30.0, 61.0, 62.0, 45.0, 59.0, 119.0),
         (10.0, 13.0, 16.0, 30.0, 33.0, 23.0))
_RESO = 416.0
_NWS = (13, 26, 52)
_BASES = (0, 507, 2535)
_NANS = (507, 2028, 8112)
_A = 10647
_NT = 50
_NTP = 64
_NB = 16
_TH = 0.6
_GD = 16


def _iou(x1, y1, w1, h1, x2, y2, w2, h2):
    ax1 = x1 - w1 / 2.0
    ax2 = x1 + w1 / 2.0
    ay1 = y1 - h1 / 2.0
    ay2 = y1 + h1 / 2.0
    bx1 = x2 - w2 / 2.0
    bx2 = x2 + w2 / 2.0
    by1 = y2 - h2 / 2.0
    by2 = y2 + h2 / 2.0
    iw = jnp.maximum(jnp.minimum(ax2, bx2) - jnp.maximum(ax1, bx1), 0.0)
    ih = jnp.maximum(jnp.minimum(ay2, by2) - jnp.maximum(ay1, by1), 0.0)
    inter = iw * ih
    union = (ax2 - ax1) * (ay2 - ay1) + (bx2 - bx1) * (by2 - by1) - inter
    return inter / jnp.maximum(union, 1e-16)


def _best_slot(t1, t2, t3, t4, lvl):
    nw = float(_NWS[lvl])
    nwi = _NWS[lvl]
    nan = _NANS[lvl]
    aw = _ANCH[lvl]
    gw = t3 * nw
    gh = t4 * nw
    i0 = _iou(0.0, 0.0, aw[0], aw[1], 0.0, 0.0, gw, gh)
    i1 = _iou(0.0, 0.0, aw[2], aw[3], 0.0, 0.0, gw, gh)
    i2 = _iou(0.0, 0.0, aw[4], aw[5], 0.0, 0.0, gw, gh)
    bn = jnp.where((i0 >= i1) & (i0 >= i2), 0, jnp.where(i1 >= i2, 1, 2))
    bmax = jnp.maximum(jnp.maximum(i0, i1), i2)
    bn = jnp.where(bmax > 0.0, bn, -1)
    gi = (t1 * nw).astype(jnp.int32)
    gj = (t2 * nw).astype(jnp.int32)
    idx = 3 * (nwi * gi + gj) + bn
    return jnp.where(idx < 0, idx + nan, idx)


def _sc_gather_kernel(tgt_hbm, table_hbm, g_hbm, tv, idxv, rows, sem):
    wid = lax.axis_index("s") * 2 + lax.axis_index("c")

    @pl.when(wid < _NB)
    def _():
        b = wid
        pltpu.sync_copy(tgt_hbm.at[b], tv)
        for lvl in range(3):
            for i in range(_NTP // 16):
                sl = pl.ds(i * 16, 16)
                t1 = tv[1, sl]
                t2 = tv[2, sl]
                t3 = tv[3, sl]
                t4 = tv[4, sl]
                idx = _best_slot(t1, t2, t3, t4, lvl)
                g = idx + (b * _A + _BASES[lvl])
                idxv[lvl, sl] = lax.shift_right_logical(g, 3)
        copies = [
            pltpu.async_copy(table_hbm.at[idxv.at[lvl]], rows.at[lvl], sem)
            for lvl in range(3)
        ]
        for c in copies:
            c.wait()
        pltpu.sync_copy(rows, g_hbm.at[b])


def _sc_gather(trt64, out16):
    mesh = plsc.VectorSubcoreMesh(core_axis_name="c", subcore_axis_name="s")
    run = functools.partial(
        pl.kernel,
        mesh=mesh,
        out_type=jax.ShapeDtypeStruct((_NB, 3, _NTP, 128), jnp.float32),
        scratch_types=[
            pltpu.VMEM((5, _NTP), jnp.float32),
            pltpu.VMEM((3, _NTP), jnp.int32),
            pltpu.VMEM((3, _NTP, 128), jnp.float32),
            pltpu.SemaphoreType.DMA,
        ],
    )(_sc_gather_kernel)
    return run(trt64, out16)


def _corners(x, y, w, h):
    x1 = x - w / 2.0
    x2 = x + w / 2.0
    y1 = y - h / 2.0
    y2 = y + h / 2.0
    return x1, x2, y1, y2, (x2 - x1) * (y2 - y1)


_INVW = (1.0 / 13.0, 1.0 / 26.0, 1.0 / 52.0)


def _body(ot_ref, trc_ref, trt_ref, g_ref, loss_ref):
    px = ot_ref[0, 0:1, :] / _RESO
    py = ot_ref[0, 1:2, :] / _RESO
    pw = ot_ref[0, 2:3, :] / _RESO
    ph = ot_ref[0, 3:4, :] / _RESO
    base = 0.5 * (jnp.sum((px - 0.5) ** 2) + jnp.sum((py - 0.5) ** 2)
                  + jnp.sum(pw * pw) + jnp.sum(ph * ph))
    cls = ot_ref[0, 5:85, :]
    base = base - jnp.sum(jnp.maximum(jnp.log(1.0 - cls), -100.0))
    conf = ot_ref[0, 5:6, :]

    t0r = trt_ref[0, 0:1, :]
    t1r = trt_ref[0, 1:2, :]
    t2r = trt_ref[0, 2:3, :]
    t3r = trt_ref[0, 3:4, :]
    t4r = trt_ref[0, 4:5, :]
    trc = trc_ref[0]
    t0c = trc[:, 0:1]
    t1c = trc[:, 1:2]
    t2c = trc[:, 2:3]
    t3c = trc[:, 3:4]
    t4c = trc[:, 4:5]

    ii = lax.broadcasted_iota(jnp.int32, (_NT, _NT), 0)
    jj = lax.broadcasted_iota(jnp.int32, (_NT, _NT), 1)
    zr = (t1r == 0.0).astype(jnp.float32)
    zc = (t1c == 0.0).astype(jnp.float32)
    prefT = jnp.sum(jnp.where(ii <= jj, zc, 0.0), axis=0, keepdims=True)
    aliveT = prefT == 0.0
    prefC = jnp.sum(jnp.where(jj <= ii, zr, 0.0), axis=1, keepdims=True)
    aliveC = prefC == 0.0

    li = lax.broadcasted_iota(jnp.int32, (1, _A), 1)
    invnw = jnp.where(li < _BASES[1], _INVW[0],
                      jnp.where(li < _BASES[2], _INVW[1], _INVW[2]))
    ax1, ax2, ay1, ay2, area_a = _corners(px * invnw, py * invnw,
                                          pw * invnw, ph * invnw)
    bx1, bx2, by1, by2, area_b = _corners(t1c, t2c, t3c, t4c)
    iw = jnp.maximum(jnp.minimum(ax2, bx2) - jnp.maximum(ax1, bx1), 0.0)
    ih = jnp.maximum(jnp.minimum(ay2, by2) - jnp.maximum(ay1, by1), 0.0)
    inter = iw * ih
    iou_all = inter / jnp.maximum(area_a + area_b - inter, 1e-16)
    iou_all = jnp.where(aliveC, iou_all, 0.0)
    cur = jnp.max(iou_all, axis=0, keepdims=True)
    base = base + jnp.sum(jnp.where(cur > _TH, 0.0, 12.5 * conf * conf))

    iota128 = lax.broadcasted_iota(jnp.int32, (_NT, 128), 1)
    b_id = pl.program_id(0)
    corr = jnp.float32(0.0)
    for lvl in range(3):
        nw = float(_NWS[lvl])
        idxr = _best_slot(t1r, t2r, t3r, t4r, lvl)
        idxc = _best_slot(t1c, t2c, t3c, t4c, lvl)

        eq = idxc == idxr
        later = jnp.sum(jnp.where((jj > ii) & eq & aliveT, 1.0, 0.0),
                        axis=1, keepdims=True)
        winc = aliveC & (later == 0.0)
        keyc = idxc * 128 + t0c.astype(jnp.int32)
        keyr = idxr * 128 + t0r.astype(jnp.int32)
        eqk = keyc == keyr
        earlier = jnp.sum(jnp.where((jj < ii) & eqk & aliveT, 1.0, 0.0),
                          axis=1, keepdims=True)
        firstc = aliveC & (earlier == 0.0)

        g = g_ref[0, lvl, 0:_NT, :]
        colb = ((b_id * _A + _BASES[lvl] + idxc) & 7) * 16

        def _pick(ch):
            return jnp.sum(jnp.where(iota128 == colb + ch, g, 0.0),
                           axis=1, keepdims=True)

        pxs = _pick(0) / _RESO
        pys = _pick(1) / _RESO
        pws = _pick(2) / _RESO
        phs = _pick(3) / _RESO
        confs = _pick(5)
        ccol = t0c.astype(jnp.int32)
        vcls = _pick(5 + ccol)

        dcoord = 0.5 * ((pxs - t1c) ** 2 - (pxs - 0.5) ** 2
                        + (pys - t2c) ** 2 - (pys - 0.5) ** 2
                        + (pws - t3c) ** 2 - pws * pws
                        + (phs - t4c) ** 2 - phs * phs)
        tconf = _iou(t1c * nw, t2c * nw, t3c * nw, t4c * nw, pxs, pys, pws, phs)
        invl = _INVW[lvl]
        sx1, sx2, sy1, sy2, area_s = _corners(pxs * invl, pys * invl,
                                              pws * invl, phs * invl)
        rx1, rx2, ry1, ry2, area_r = _corners(t1r, t2r, t3r, t4r)
        siw = jnp.maximum(jnp.minimum(sx2, rx2) - jnp.maximum(sx1, rx1), 0.0)
        sih = jnp.maximum(jnp.minimum(sy2, ry2) - jnp.maximum(sy1, ry1), 0.0)
        sinter = siw * sih
        slot_all = sinter / jnp.maximum(area_s + area_r - sinter, 1e-16)
        slot_all = jnp.where(aliveT, slot_all, 0.0)
        curs = jnp.max(slot_all, axis=1, keepdims=True)
        dconf = 0.5 * (confs - tconf) ** 2 - jnp.where(
            curs > _TH, 0.0, 12.5 * confs * confs)
        dcls = (-jnp.maximum(jnp.log(vcls), -100.0)
                + jnp.maximum(jnp.log(1.0 - vcls), -100.0))
        corr = corr + jnp.sum(jnp.where(winc, dcoord + dconf, 0.0))
        corr = corr + jnp.sum(jnp.where(firstc, dcls, 0.0))

    loss_ref[0, 0, 0] = base + corr


def kernel(output, target):
    trc = target.reshape(_NB, _NT, 5)
    trt = jnp.transpose(trc, (0, 2, 1))
    trt64 = jnp.pad(trt, ((0, 0), (0, 0), (0, _NTP - _NT)))
    out16 = output[:, :, 0:_GD].reshape(_NB * _A * _GD // 128, 128)
    ot = jnp.transpose(output, (0, 2, 1))
    g = _sc_gather(trt64, out16)
    partial = pl.pallas_call(
        _body,
        grid=(_NB,),
        in_specs=[
            pl.BlockSpec((1, 85, _A), lambda b: (b, 0, 0)),
            pl.BlockSpec((1, _NT, 5), lambda b: (b, 0, 0)),
            pl.BlockSpec((1, 5, _NT), lambda b: (b, 0, 0)),
            pl.BlockSpec((1, 3, _NTP, 128), lambda b: (b, 0, 0, 0)),
        ],
        out_specs=pl.BlockSpec((1, 1, 1), lambda b: (b, 0, 0),
                               memory_space=pltpu.SMEM),
        out_shape=jax.ShapeDtypeStruct((_NB, 1, 1), jnp.float32),
    )(ot, trc, trt, g)
    return jnp.sum(partial)
